# baseline (speedup 1.0000x reference)
import functools

import jax
import jax.numpy as jnp
import numpy as np
from jax import lax
from jax.experimental import pallas as pl
from jax.experimental.pallas import tpu as pltpu
from jax.experimental.pallas import tpu_sc as plsc

NC, NS = 2, 16
NW = NC * NS
B = 128


@functools.partial(jax.jit, static_argnums=(2,))
def _sc_degree(dstp, zeros1d, NB):
    NP = zeros1d.shape[0]
    mesh = plsc.VectorSubcoreMesh(core_axis_name="c", subcore_axis_name="s")

    @functools.partial(
        pl.kernel,
        out_type=jax.ShapeDtypeStruct((NW * NP,), jnp.float32),
        mesh=mesh,
        scratch_types=[
            pltpu.VMEM((NB, B), jnp.int32),
            pltpu.VMEM((NP,), jnp.float32),
        ],
        compiler_params=pltpu.CompilerParams(needs_layout_passes=False),
    )
    def k(dst_hbm, zeros_hbm, degp_hbm, dst_v, hist):
        c = lax.axis_index("c")
        s = lax.axis_index("s")
        wid = s * NC + c
        pltpu.sync_copy(zeros_hbm, hist)
        pltpu.sync_copy(dst_hbm.at[wid], dst_v)
        ones16 = jnp.full((16,), 1.0, jnp.float32)

        def body(t, carry):
            j = t // 8
            kk = t % 8
            v = dst_v[j, pl.ds(kk * 16, 16)]
            plsc.addupdate_scatter(hist, [v], ones16)
            return carry

        lax.fori_loop(jnp.int32(0), jnp.int32(NB * 8), body, jnp.int32(0))
        pltpu.sync_copy(hist, degp_hbm.at[pl.ds(wid * NP, NP)])

    return k(dstp, zeros1d)


@functools.partial(jax.jit, static_argnums=(3,))
def _sc_scatter(h2, srcp, dstp, NB):
    NP, D = h2.shape
    PR = 40
    rps = NP // NS
    mesh = plsc.VectorSubcoreMesh(core_axis_name="c", subcore_axis_name="s")

    @functools.partial(
        pl.kernel,
        out_type=jax.ShapeDtypeStruct((NC, NP, D), jnp.float32),
        mesh=mesh,
        scratch_types=[
            pltpu.VMEM((PR, B), jnp.int32),
            pltpu.VMEM((PR, B), jnp.int32),
            pltpu.VMEM((B, D), jnp.float32),
            pltpu.VMEM((B, D), jnp.float32),
            pltpu.VMEM_SHARED((NP, D), jnp.float32),
            pltpu.SemaphoreType.DMA,
            pltpu.SemaphoreType.DMA,
        ],
    )
    def k(h2_hbm, src_hbm, dst_hbm, outp_hbm, src_v, dst_v, buf0, buf1,
          acc, sg0, sg1):
        c = lax.axis_index("c")
        s = lax.axis_index("s")
        wid = s * NC + c
        r0 = s * rps
        pltpu.sync_copy(h2_hbm.at[pl.ds(r0, rps)], acc.at[pl.ds(r0, rps)])

        def g_start(j, buf, sem):
            pltpu.async_copy(h2_hbm.at[src_v.at[j]], buf, sem)

        def g_wait(j, buf, sem):
            pltpu.make_async_copy(h2_hbm.at[src_v.at[j]], buf, sem).wait()

        for ph, cnt in enumerate([PR, NB - PR]):
            pltpu.sync_copy(src_hbm.at[wid, pl.ds(ph * PR, cnt)],
                            src_v.at[pl.ds(0, cnt)])
            pltpu.sync_copy(dst_hbm.at[wid, pl.ds(ph * PR, cnt)],
                            dst_v.at[pl.ds(0, cnt)])
            if ph == 0:
                plsc.subcore_barrier()
            T = cnt // 2
            g_start(jnp.int32(0), buf0, sg0)

            def body(t, carry):
                jj0 = 2 * t
                jj1 = jj0 + 1
                g_start(jj1, buf1, sg1)
                g_wait(jj0, buf0, sg0)
                pltpu.sync_copy(buf0, acc.at[dst_v.at[jj0]], add=True)

                @pl.when(t < T - 1)
                def _pre():
                    g_start(jj0 + 2, buf0, sg0)

                g_wait(jj1, buf1, sg1)
                pltpu.sync_copy(buf1, acc.at[dst_v.at[jj1]], add=True)
                return carry

            lax.fori_loop(jnp.int32(0), jnp.int32(T), body, jnp.int32(0))

        plsc.subcore_barrier()
        pltpu.sync_copy(acc.at[pl.ds(r0, rps)], outp_hbm.at[c, pl.ds(r0, rps)])

    return k(h2, srcp, dstp)


def _tc_h2(xp, W, degp3):
    NP, Din = xp.shape
    Dout = W.shape[1]
    BN = 1024
    NRB = BN // B
    z = np.int32(0)

    def kern(x_ref, w_ref, d_ref, h2_ref, dis_ref):
        d = jnp.sum(d_ref[...], axis=0) + 1.0
        dis = lax.rsqrt(d)
        h = jnp.dot(x_ref[...], w_ref[...], preferred_element_type=jnp.float32)
        eye = (lax.broadcasted_iota(jnp.int32, (B, B), 0)
               == lax.broadcasted_iota(jnp.int32, (B, B), 1)).astype(jnp.float32)
        for q in range(NRB):
            col = lax.dot_general(
                eye, dis[q:q + 1, :], (((1,), (1,)), ((), ())),
                preferred_element_type=jnp.float32)
            h2_ref[q * B:(q + 1) * B, :] = h[q * B:(q + 1) * B, :] * col
            dis_ref[q * B:(q + 1) * B, :] = col

    return pl.pallas_call(
        kern,
        grid=(NP // BN,),
        in_specs=[
            pl.BlockSpec((BN, Din), lambda i: (i, z)),
            pl.BlockSpec((Din, Dout), lambda i: (z, z)),
            pl.BlockSpec((NW, NRB, B), lambda i: (z, i, z)),
        ],
        out_specs=[
            pl.BlockSpec((BN, Dout), lambda i: (i, z)),
            pl.BlockSpec((BN, 1), lambda i: (i, z)),
        ],
        out_shape=[
            jax.ShapeDtypeStruct((NP, Dout), jnp.float32),
            jax.ShapeDtypeStruct((NP, 1), jnp.float32),
        ],
    )(xp, W, degp3)


def _tc_combine(p, h2, dis, b, N):
    D = h2.shape[1]
    BN = 1000
    z = np.int32(0)

    def kern(p_ref, h2_ref, dis_ref, b_ref, o_ref):
        agg = p_ref[0] + p_ref[1] - h2_ref[...]
        o_ref[...] = jnp.maximum(agg * dis_ref[...] + b_ref[...], 0.0)

    return pl.pallas_call(
        kern,
        grid=(N // BN,),
        in_specs=[
            pl.BlockSpec((NC, BN, D), lambda i: (z, i, z)),
            pl.BlockSpec((BN, D), lambda i: (i, z)),
            pl.BlockSpec((BN, 1), lambda i: (i, z)),
            pl.BlockSpec((1, D), lambda i: (z, z)),
        ],
        out_specs=pl.BlockSpec((BN, D), lambda i: (i, z)),
        out_shape=jax.ShapeDtypeStruct((N, D), jnp.float32),
    )(p, h2, dis, b.reshape(1, D))


def kernel(x, edge_index, W, b):
    N, Din = x.shape
    E = edge_index.shape[1]
    x = x.astype(jnp.float32)
    W = W.astype(jnp.float32)
    b = b.astype(jnp.float32)

    src = edge_index[0].astype(jnp.int32)
    dst = edge_index[1].astype(jnp.int32)

    EW = E // NW
    NB = 2 * (-(-EW // (2 * B)))
    pad = NB * B - EW
    NP = (-(-N // (NS * 8)) + 1) * (NS * 8)

    srcp = jnp.concatenate(
        [src.reshape(NW, EW), jnp.zeros((NW, pad), jnp.int32)], axis=1
    ).reshape(NW, NB, B)
    dump = N + (jnp.arange(pad, dtype=jnp.int32) % (NP - N))
    dstp = jnp.concatenate(
        [dst.reshape(NW, EW), jnp.broadcast_to(dump[None, :], (NW, pad))], axis=1
    ).reshape(NW, NB, B)

    xp = jnp.concatenate([x, jnp.zeros((NP - N, Din), jnp.float32)], axis=0)
    zeros1d = jnp.zeros((NP,), jnp.float32)

    degp3 = _sc_degree(dstp, zeros1d, NB).reshape(NW, NP // B, B)
    h2, dis = _tc_h2(xp, W, degp3)
    outp = _sc_scatter(h2, srcp, dstp, NB)
    return _tc_combine(outp, h2, dis, b, N).astype(jnp.float64)


# speedup vs baseline: 1.8271x; 1.8271x over previous
import functools

import jax
import jax.numpy as jnp
import numpy as np
from jax import lax
from jax.experimental import pallas as pl
from jax.experimental.pallas import tpu as pltpu
from jax.experimental.pallas import tpu_sc as plsc

NC, NS = 2, 16
NW = NC * NS
B = 128


@functools.partial(jax.jit, static_argnums=(2,))
def _sc_degree(dstf, zeros1d, EW):
    NP = zeros1d.shape[0]
    mesh = plsc.VectorSubcoreMesh(core_axis_name="c", subcore_axis_name="s")

    @functools.partial(
        pl.kernel,
        out_type=jax.ShapeDtypeStruct((NW * NP,), jnp.float32),
        mesh=mesh,
        scratch_types=[
            pltpu.VMEM((EW,), jnp.int32),
            pltpu.VMEM((NP,), jnp.float32),
        ],
        compiler_params=pltpu.CompilerParams(needs_layout_passes=False),
    )
    def k(dst_hbm, zeros_hbm, degp_hbm, dst_v, hist):
        c = lax.axis_index("c")
        s = lax.axis_index("s")
        wid = s * NC + c
        pltpu.sync_copy(zeros_hbm, hist)
        pltpu.sync_copy(dst_hbm.at[pl.ds(wid * EW, EW)], dst_v)
        ones16 = jnp.full((16,), 1.0, jnp.float32)

        def body(t, carry):
            v = dst_v[pl.ds(t * 16, 16)]
            plsc.addupdate_scatter(hist, [v], ones16)
            return carry

        lax.fori_loop(jnp.int32(0), jnp.int32(EW // 16), body, jnp.int32(0))
        pltpu.sync_copy(hist, degp_hbm.at[pl.ds(wid * NP, NP)])

    return k(dstf, zeros1d)


@functools.partial(jax.jit, static_argnums=(3,))
def _sc_scatter(h2, srcp, dstp, NB):
    NP, D = h2.shape
    BE = srcp.shape[2]
    rps = NP // NS
    mesh = plsc.VectorSubcoreMesh(core_axis_name="c", subcore_axis_name="s")

    @functools.partial(
        pl.kernel,
        out_type=jax.ShapeDtypeStruct((NC, NP, D), jnp.float32),
        mesh=mesh,
        scratch_types=[
            pltpu.VMEM((NB, BE), jnp.int32),
            pltpu.VMEM((NB, BE), jnp.int32),
            pltpu.VMEM((BE, D), jnp.float32),
            pltpu.VMEM_SHARED((NP, D), jnp.float32),
            pltpu.SemaphoreType.DMA,
        ],
    )
    def k(h2_hbm, src_hbm, dst_hbm, outp_hbm, src_v, dst_v, rows0, acc, sg0):
        c = lax.axis_index("c")
        s = lax.axis_index("s")
        wid = s * NC + c
        r0 = s * rps
        pltpu.sync_copy(h2_hbm.at[pl.ds(r0, rps)], acc.at[pl.ds(r0, rps)])
        pltpu.sync_copy(src_hbm.at[wid], src_v)
        pltpu.sync_copy(dst_hbm.at[wid], dst_v)
        plsc.subcore_barrier()

        def body(j, carry):
            pltpu.async_copy(h2_hbm.at[src_v.at[j]], rows0, sg0).wait()
            pltpu.sync_copy(rows0, acc.at[dst_v.at[j]], add=True)
            return carry

        lax.fori_loop(jnp.int32(0), jnp.int32(NB), body, jnp.int32(0))
        plsc.subcore_barrier()
        pltpu.sync_copy(acc.at[pl.ds(r0, rps)], outp_hbm.at[c, pl.ds(r0, rps)])

    return k(h2, srcp, dstp)


def _tc_h2(xp, W, degp3):
    NP, Din = xp.shape
    Dout = W.shape[1]
    BN = 1024
    NRB = BN // B
    z = np.int32(0)

    def kern(x_ref, w_ref, d_ref, h2_ref, dis_ref):
        d = jnp.sum(d_ref[...], axis=0) + 1.0
        dis = lax.rsqrt(d)
        h = jnp.dot(x_ref[...], w_ref[...], preferred_element_type=jnp.float32)
        eye = (lax.broadcasted_iota(jnp.int32, (B, B), 0)
               == lax.broadcasted_iota(jnp.int32, (B, B), 1)).astype(jnp.float32)
        for q in range(NRB):
            col = lax.dot_general(
                eye, dis[q:q + 1, :], (((1,), (1,)), ((), ())),
                preferred_element_type=jnp.float32)
            h2_ref[q * B:(q + 1) * B, :] = h[q * B:(q + 1) * B, :] * col
            dis_ref[q * B:(q + 1) * B, :] = col

    return pl.pallas_call(
        kern,
        grid=(NP // BN,),
        in_specs=[
            pl.BlockSpec((BN, Din), lambda i: (i, z)),
            pl.BlockSpec((Din, Dout), lambda i: (z, z)),
            pl.BlockSpec((NW, NRB, B), lambda i: (z, i, z)),
        ],
        out_specs=[
            pl.BlockSpec((BN, Dout), lambda i: (i, z)),
            pl.BlockSpec((BN, 1), lambda i: (i, z)),
        ],
        out_shape=[
            jax.ShapeDtypeStruct((NP, Dout), jnp.float32),
            jax.ShapeDtypeStruct((NP, 1), jnp.float32),
        ],
    )(xp, W, degp3)


def _tc_combine(p, h2, dis, b, N):
    D = h2.shape[1]
    BN = 1000
    z = np.int32(0)

    def kern(p_ref, h2_ref, dis_ref, b_ref, o_ref):
        agg = p_ref[0] + p_ref[1] - h2_ref[...]
        o_ref[...] = jnp.maximum(agg * dis_ref[...] + b_ref[...], 0.0)

    return pl.pallas_call(
        kern,
        grid=(N // BN,),
        in_specs=[
            pl.BlockSpec((NC, BN, D), lambda i: (z, i, z)),
            pl.BlockSpec((BN, D), lambda i: (i, z)),
            pl.BlockSpec((BN, 1), lambda i: (i, z)),
            pl.BlockSpec((1, D), lambda i: (z, z)),
        ],
        out_specs=pl.BlockSpec((BN, D), lambda i: (i, z)),
        out_shape=jax.ShapeDtypeStruct((N, D), jnp.float32),
    )(p, h2, dis, b.reshape(1, D))


def kernel(x, edge_index, W, b):
    N, Din = x.shape
    E = edge_index.shape[1]
    x = x.astype(jnp.float32)
    W = W.astype(jnp.float32)
    b = b.astype(jnp.float32)

    src = edge_index[0].astype(jnp.int32)
    dst = edge_index[1].astype(jnp.int32)

    EW = E // NW
    BE = 125
    NB = EW // BE
    NP = (-(-N // (NS * 8)) + 1) * (NS * 8)

    srcp = src.reshape(NW, NB, BE)
    dstp = dst.reshape(NW, NB, BE)

    xp = jnp.concatenate([x, jnp.zeros((NP - N, Din), jnp.float32)], axis=0)
    zeros1d = jnp.zeros((NP,), jnp.float32)

    degp3 = _sc_degree(dst, zeros1d, EW).reshape(NW, NP // B, B)
    h2, dis = _tc_h2(xp, W, degp3)
    outp = _sc_scatter(h2, srcp, dstp, NB)
    return _tc_combine(outp, h2, dis, b, N).astype(jnp.float64)


# speedup vs baseline: 2.3764x; 1.3006x over previous
import functools

import jax
import jax.numpy as jnp
import numpy as np
from jax import lax
from jax.experimental import pallas as pl
from jax.experimental.pallas import tpu as pltpu
from jax.experimental.pallas import tpu_sc as plsc

NC, NS = 2, 16
NW = NC * NS
B = 128


@functools.partial(jax.jit, static_argnums=(2,))
def _sc_degree(dstf, zeros1d, EW):
    NP = zeros1d.shape[0]
    mesh = plsc.VectorSubcoreMesh(core_axis_name="c", subcore_axis_name="s")

    @functools.partial(
        pl.kernel,
        out_type=jax.ShapeDtypeStruct((NW * NP,), jnp.float32),
        mesh=mesh,
        scratch_types=[
            pltpu.VMEM((EW,), jnp.int32),
            pltpu.VMEM((NP,), jnp.float32),
        ],
        compiler_params=pltpu.CompilerParams(needs_layout_passes=False),
    )
    def k(dst_hbm, zeros_hbm, degp_hbm, dst_v, hist):
        c = lax.axis_index("c")
        s = lax.axis_index("s")
        wid = s * NC + c
        pltpu.sync_copy(zeros_hbm, hist)
        pltpu.sync_copy(dst_hbm.at[pl.ds(wid * EW, EW)], dst_v)
        ones16 = jnp.full((16,), 1.0, jnp.float32)

        def body(t, carry):
            v = dst_v[pl.ds(t * 16, 16)]
            plsc.addupdate_scatter(hist, [v], ones16)
            return carry

        lax.fori_loop(jnp.int32(0), jnp.int32(EW // 16), body, jnp.int32(0))
        pltpu.sync_copy(hist, degp_hbm.at[pl.ds(wid * NP, NP)])

    return k(dstf, zeros1d)


@functools.partial(jax.jit, static_argnums=(3,))
def _sc_scatter(h2, srcp, dstp, NB):
    NP, D = h2.shape
    BE = srcp.shape[2]
    rps = NP // NS
    mesh = plsc.VectorSubcoreMesh(core_axis_name="c", subcore_axis_name="s")

    @functools.partial(
        pl.kernel,
        out_type=jax.ShapeDtypeStruct((NC, NP, D), jnp.float32),
        mesh=mesh,
        scratch_types=[
            pltpu.VMEM((NB // 2, BE), jnp.int32),
            pltpu.VMEM((NB // 2, BE), jnp.int32),
            pltpu.VMEM((BE, D), jnp.float32),
            pltpu.VMEM((BE, D), jnp.float32),
            pltpu.VMEM_SHARED((NP, D), jnp.float32),
            pltpu.SemaphoreType.DMA,
            pltpu.SemaphoreType.DMA,
        ],
    )
    def k(h2_hbm, src_hbm, dst_hbm, outp_hbm, src_v, dst_v, buf0, buf1,
          acc, sg0, sg1):
        c = lax.axis_index("c")
        s = lax.axis_index("s")
        wid = s * NC + c
        r0 = s * rps
        pltpu.sync_copy(h2_hbm.at[pl.ds(r0, rps)], acc.at[pl.ds(r0, rps)])

        def g_start(j, buf, sem):
            pltpu.async_copy(h2_hbm.at[src_v.at[j]], buf, sem)

        def g_wait(j, buf, sem):
            pltpu.make_async_copy(h2_hbm.at[src_v.at[j]], buf, sem).wait()

        PR = NB // 2
        for ph in range(2):
            pltpu.sync_copy(src_hbm.at[wid, pl.ds(ph * PR, PR)], src_v)
            pltpu.sync_copy(dst_hbm.at[wid, pl.ds(ph * PR, PR)], dst_v)
            if ph == 0:
                plsc.subcore_barrier()
            T = PR // 2
            g_start(jnp.int32(0), buf0, sg0)

            def body(t, carry):
                jj0 = 2 * t
                jj1 = jj0 + 1
                g_start(jj1, buf1, sg1)
                g_wait(jj0, buf0, sg0)
                pltpu.sync_copy(buf0, acc.at[dst_v.at[jj0]], add=True)

                @pl.when(t < T - 1)
                def _pre():
                    g_start(jj0 + 2, buf0, sg0)

                g_wait(jj1, buf1, sg1)
                pltpu.sync_copy(buf1, acc.at[dst_v.at[jj1]], add=True)
                return carry

            lax.fori_loop(jnp.int32(0), jnp.int32(T), body, jnp.int32(0))

        plsc.subcore_barrier()
        pltpu.sync_copy(acc.at[pl.ds(r0, rps)], outp_hbm.at[c, pl.ds(r0, rps)])

    return k(h2, srcp, dstp)


def _tc_h2(xp, W, degp3):
    NP, Din = xp.shape
    Dout = W.shape[1]
    BN = 1024
    NRB = BN // B
    z = np.int32(0)

    def kern(x_ref, w_ref, d_ref, h2_ref, dis_ref):
        d = jnp.sum(d_ref[...], axis=0) + 1.0
        dis = lax.rsqrt(d)
        h = jnp.dot(x_ref[...], w_ref[...], preferred_element_type=jnp.float32)
        eye = (lax.broadcasted_iota(jnp.int32, (B, B), 0)
               == lax.broadcasted_iota(jnp.int32, (B, B), 1)).astype(jnp.float32)
        for q in range(NRB):
            col = lax.dot_general(
                eye, dis[q:q + 1, :], (((1,), (1,)), ((), ())),
                preferred_element_type=jnp.float32)
            h2_ref[q * B:(q + 1) * B, :] = h[q * B:(q + 1) * B, :] * col
            dis_ref[q * B:(q + 1) * B, :] = col

    return pl.pallas_call(
        kern,
        grid=(NP // BN,),
        in_specs=[
            pl.BlockSpec((BN, Din), lambda i: (i, z)),
            pl.BlockSpec((Din, Dout), lambda i: (z, z)),
            pl.BlockSpec((NW, NRB, B), lambda i: (z, i, z)),
        ],
        out_specs=[
            pl.BlockSpec((BN, Dout), lambda i: (i, z)),
            pl.BlockSpec((BN, 1), lambda i: (i, z)),
        ],
        out_shape=[
            jax.ShapeDtypeStruct((NP, Dout), jnp.float32),
            jax.ShapeDtypeStruct((NP, 1), jnp.float32),
        ],
    )(xp, W, degp3)


def _tc_combine(p, h2, dis, b, N):
    D = h2.shape[1]
    BN = 1000
    z = np.int32(0)

    def kern(p_ref, h2_ref, dis_ref, b_ref, o_ref):
        agg = p_ref[0] + p_ref[1] - h2_ref[...]
        o_ref[...] = jnp.maximum(agg * dis_ref[...] + b_ref[...], 0.0)

    return pl.pallas_call(
        kern,
        grid=(N // BN,),
        in_specs=[
            pl.BlockSpec((NC, BN, D), lambda i: (z, i, z)),
            pl.BlockSpec((BN, D), lambda i: (i, z)),
            pl.BlockSpec((BN, 1), lambda i: (i, z)),
            pl.BlockSpec((1, D), lambda i: (z, z)),
        ],
        out_specs=pl.BlockSpec((BN, D), lambda i: (i, z)),
        out_shape=jax.ShapeDtypeStruct((N, D), jnp.float32),
    )(p, h2, dis, b.reshape(1, D))


def kernel(x, edge_index, W, b):
    N, Din = x.shape
    E = edge_index.shape[1]
    x = x.astype(jnp.float32)
    W = W.astype(jnp.float32)
    b = b.astype(jnp.float32)

    src = edge_index[0].astype(jnp.int32)
    dst = edge_index[1].astype(jnp.int32)

    EW = E // NW
    BE = 125
    NB = EW // BE
    NP = (-(-N // (NS * 8)) + 1) * (NS * 8)

    srcp = src.reshape(NW, NB, BE)
    dstp = dst.reshape(NW, NB, BE)

    xp = jnp.concatenate([x, jnp.zeros((NP - N, Din), jnp.float32)], axis=0)
    zeros1d = jnp.zeros((NP,), jnp.float32)

    degp3 = _sc_degree(dst, zeros1d, EW).reshape(NW, NP // B, B)
    h2, dis = _tc_h2(xp, W, degp3)
    outp = _sc_scatter(h2, srcp, dstp, NB)
    return _tc_combine(outp, h2, dis, b, N).astype(jnp.float64)
